# baseline (speedup 1.0000x reference)
import functools

import jax
import jax.numpy as jnp
from jax import lax
from jax.experimental import pallas as pl
from jax.experimental.pallas import tpu as pltpu
from jax.experimental.pallas import tpu_sc as plsc

N = 10000
NP = 10240
E = 320000
EP = 327680
D_IN = 128
D_HID = 128
D_LAT = 64
NC = 2
NS = 16
NW = NC * NS
CH = 128
NCH = EP // (NW * CH)
RPT = NP // NS
NPA = NP
RPTA = NPA // NS
NCHS = EP // (NS * CH)
ZB = 64
DUMMY = N
BLK = 10240


def _sc_mesh():
    return plsc.VectorSubcoreMesh(core_axis_name="c", subcore_axis_name="s")


@functools.partial(
    pl.kernel,
    out_type=jax.ShapeDtypeStruct((NC, NP, 16), jnp.float32),
    mesh=_sc_mesh(),
    scratch_types=[
        pltpu.VMEM((NCH, CH), jnp.int32),
        pltpu.VMEM((CH, 16), jnp.float32),
        pltpu.VMEM((ZB, 16), jnp.float32),
        pltpu.VMEM_SHARED((NP, 16), jnp.float32),
    ],
    compiler_params=pltpu.CompilerParams(use_tc_tiling_on_sc=False),
)
def _deg_kernel(dst_hbm, out_hbm, dst_v, ones_v, zb_v, acc_sh):
    c = lax.axis_index("c")
    s = lax.axis_index("s")
    wid = s * NC + c
    pltpu.sync_copy(dst_hbm.at[wid], dst_v)
    one16 = jnp.ones((16,), jnp.float32)
    zero16 = jnp.zeros((16,), jnp.float32)

    def fill_ones(i, _):
        ones_v[i, :] = one16
        return 0

    lax.fori_loop(0, CH, fill_ones, 0)

    def fill_zero(i, _):
        zb_v[i, :] = zero16
        return 0

    lax.fori_loop(0, ZB, fill_zero, 0)

    base = s * RPT

    def zcopy(i, _):
        pltpu.sync_copy(zb_v, acc_sh.at[pl.ds(base + i * ZB, ZB)])
        return 0

    lax.fori_loop(0, RPT // ZB, zcopy, 0)
    plsc.subcore_barrier()

    def chunk(j, _):
        pltpu.sync_copy(ones_v, acc_sh.at[dst_v.at[j]], add=True)
        return 0

    lax.fori_loop(0, NCH, chunk, 0)
    plsc.subcore_barrier()
    pltpu.sync_copy(acc_sh.at[pl.ds(base, RPT)], out_hbm.at[c, pl.ds(base, RPT)])


def _make_scatter(DH):
    @functools.partial(
        pl.kernel,
        out_type=jax.ShapeDtypeStruct((NC, NP, DH), jnp.float32),
        mesh=_sc_mesh(),
        scratch_types=[
            pltpu.VMEM((NCHS, CH), jnp.int32),
            pltpu.VMEM((NCHS, CH), jnp.int32),
            [pltpu.VMEM((CH, DH), jnp.float32)] * 4,
            [pltpu.VMEM((CH,), jnp.int32)] * 4,
            pltpu.VMEM((CH,), jnp.int32),
            pltpu.VMEM((ZB, DH), jnp.float32),
            pltpu.VMEM_SHARED((NPA, DH), jnp.float32),
            [pltpu.SemaphoreType.DMA] * 4,
        ],
        compiler_params=pltpu.CompilerParams(use_tc_tiling_on_sc=False),
    )
    def _scatter(hsa_hbm, hsb_hbm, src_hbm, dst_hbm, out_hbm, src_v, dst_v,
                 rows_v, gidx, sidx, zb_v, acc_sh, gsem):
        c = lax.axis_index("c")
        s = lax.axis_index("s")
        pltpu.sync_copy(src_hbm.at[s], src_v)
        pltpu.sync_copy(dst_hbm.at[s], dst_v)
        zero16 = jnp.zeros((16,), jnp.float32)

        def fz(i, _):
            for l in range(DH // 16):
                zb_v[i, pl.ds(l * 16, 16)] = zero16
            return 0

        lax.fori_loop(0, ZB, fz, 0)
        base = s * RPTA

        def zc(i, _):
            pltpu.sync_copy(zb_v, acc_sh.at[pl.ds(base + i * ZB, ZB)])
            return 0

        lax.fori_loop(0, RPTA // ZB, zc, 0)
        if RPTA % ZB:
            pltpu.sync_copy(zb_v.at[pl.ds(0, RPTA % ZB)],
                            acc_sh.at[pl.ds(base + (RPTA // ZB) * ZB,
                                            RPTA % ZB)])
        plsc.subcore_barrier()

        def _cpidx(dst_ref, src_row):
            for k in range(CH // 16):
                dst_ref[pl.ds(k * 16, 16)] = src_row[pl.ds(k * 16, 16)]

        def _gather(b):
            @pl.when(c == 0)
            def _():
                pltpu.async_copy(hsa_hbm.at[gidx[b]], rows_v[b], gsem[b])

            @pl.when(c == 1)
            def _():
                pltpu.async_copy(hsb_hbm.at[gidx[b]], rows_v[b], gsem[b])

        for b in range(3):
            _cpidx(gidx[b], src_v.at[b])
            _gather(b)

        def rnd(jo, _):
            for b in range(4):
                j = jo * 4 + b
                pltpu.make_async_copy(hsa_hbm.at[gidx[b]], rows_v[b],
                                      gsem[b]).wait()

                @pl.when(j + 3 < NCHS)
                def _(b=b, j=j):
                    nb = (b + 3) % 4
                    _cpidx(gidx[nb], src_v.at[j + 3])
                    _gather(nb)

                _cpidx(sidx, dst_v.at[j])
                pltpu.sync_copy(rows_v[b], acc_sh.at[sidx], add=True)
            return 0

        lax.fori_loop(0, NCHS // 4, rnd, 0)
        plsc.subcore_barrier()
        pltpu.sync_copy(acc_sh.at[pl.ds(base, RPTA)],
                        out_hbm.at[c, pl.ds(base, RPTA)])

    return _scatter


_scatter128 = _make_scatter(D_HID // 2)
_scatter64 = _make_scatter(D_LAT // 2)


@functools.partial(
    pl.kernel,
    out_type=jax.ShapeDtypeStruct((NW, NCH, CH), jnp.float32),
    mesh=_sc_mesh(),
    scratch_types=[
        pltpu.VMEM((NCH, CH), jnp.int32),
        pltpu.VMEM((NCH, CH), jnp.int32),
        [pltpu.VMEM((CH, D_LAT), jnp.float32)] * 2,
        [pltpu.VMEM((CH, D_LAT), jnp.float32)] * 2,
        [pltpu.VMEM((CH,), jnp.int32)] * 2,
        [pltpu.VMEM((CH,), jnp.int32)] * 2,
        [pltpu.VMEM((CH,), jnp.float32)] * 2,
        pltpu.VMEM((80,), jnp.float32),
        [pltpu.SemaphoreType.DMA] * 2,
        [pltpu.SemaphoreType.DMA] * 2,
        [pltpu.SemaphoreType.DMA] * 2,
    ],
    compiler_params=pltpu.CompilerParams(use_tc_tiling_on_sc=False,
                                         needs_layout_passes=False),
)
def _scorer(a_hbm, b_hbm, esrc_hbm, edst_hbm, wc_hbm, out_hbm,
            src_v, dst_v, ar_v, br_v, si_v, di_v, res_v, wc_v,
            sema, semb, semo):
    c = lax.axis_index("c")
    s = lax.axis_index("s")
    wid = s * NC + c
    pltpu.sync_copy(esrc_hbm.at[wid], src_v)
    pltpu.sync_copy(edst_hbm.at[wid], dst_v)
    pltpu.sync_copy(wc_hbm, wc_v)
    lane = lax.broadcasted_iota(jnp.int32, (16,), 0)
    b2s = plsc.load_gather(wc_v, [jnp.full((16,), 64, jnp.int32)])
    wregs = [wc_v[pl.ds(k * 16, 16)] for k in range(4)]
    eqm = [lane == u for u in range(16)]

    def _cpidx(dst_ref, src_row):
        for k in range(CH // 16):
            dst_ref[pl.ds(k * 16, 16)] = src_row[pl.ds(k * 16, 16)]

    NB = 2
    for b in range(NB):
        _cpidx(si_v[b], src_v.at[b])
        _cpidx(di_v[b], dst_v.at[b])
        pltpu.async_copy(a_hbm.at[si_v[b]], ar_v[b], sema[b])
        pltpu.async_copy(b_hbm.at[di_v[b]], br_v[b], semb[b])

    def rnd(jo, _):
        for b in range(NB):
            j = jo * NB + b
            pltpu.make_async_copy(a_hbm.at[si_v[b]], ar_v[b],
                                  sema[b]).wait()
            pltpu.make_async_copy(b_hbm.at[di_v[b]], br_v[b],
                                  semb[b]).wait()
            arb = ar_v[b]
            brb = br_v[b]

            @pl.when(jo > 0)
            def _(b=b):
                pltpu.make_async_copy(res_v[b], out_hbm.at[wid, 0],
                                      semo[b]).wait()

            def group(g, _, arb=arb, brb=brb, b=b):
                vres = b2s
                for u in range(16):
                    e = g * 16 + u
                    t = None
                    for k in range(4):
                        av = arb[e, pl.ds(k * 16, 16)]
                        bv = brb[e, pl.ds(k * 16, 16)]
                        tk = jnp.maximum(av + bv, 0.0) * wregs[k]
                        t = tk if t is None else t + tk
                    s = jnp.sum(t)
                    vres = jnp.where(eqm[u], jnp.full((16,), s, jnp.float32),
                                     vres)
                res_v[b][pl.ds(g * 16, 16)] = 1.0 / (1.0 + jnp.exp(-vres))
                return 0

            lax.fori_loop(0, CH // 16, group, 0)
            pltpu.async_copy(res_v[b], out_hbm.at[wid, j], semo[b])

            @pl.when(jo + 1 < NCH // NB)
            def _(b=b, jo=jo):
                j2 = (jo + 1) * NB + b
                _cpidx(si_v[b], src_v.at[j2])
                _cpidx(di_v[b], dst_v.at[j2])
                pltpu.async_copy(a_hbm.at[si_v[b]], ar_v[b], sema[b])
                pltpu.async_copy(b_hbm.at[di_v[b]], br_v[b], semb[b])

        return 0

    lax.fori_loop(0, NCH // NB, rnd, 0)
    for b in range(NB):
        pltpu.make_async_copy(res_v[b], out_hbm.at[wid, 0], semo[b]).wait()


def _tcb_body(x_ref, w1_ref, d0_ref, d1_ref, hsa_ref, hsb_ref, dis_ref):
    deg = d0_ref[:, 0:1] + d1_ref[:, 0:1] + 1.0
    dis = lax.rsqrt(deg)
    h1 = jnp.dot(x_ref[...], w1_ref[...], preferred_element_type=jnp.float32)
    hs = h1 * dis
    hsa_ref[...] = hs[:, :D_HID // 2]
    hsb_ref[...] = hs[:, D_HID // 2:]
    dis_ref[...] = dis


_tcb = pl.pallas_call(
    _tcb_body,
    grid=(NP // BLK,),
    in_specs=[
        pl.BlockSpec((BLK, D_IN), lambda i: (i, 0)),
        pl.BlockSpec((D_IN, D_HID), lambda i: (0, 0)),
        pl.BlockSpec((BLK, 16), lambda i: (i, 0)),
        pl.BlockSpec((BLK, 16), lambda i: (i, 0)),
    ],
    out_specs=[
        pl.BlockSpec((BLK, D_HID // 2), lambda i: (i, 0)),
        pl.BlockSpec((BLK, D_HID // 2), lambda i: (i, 0)),
        pl.BlockSpec((BLK, 1), lambda i: (i, 0)),
    ],
    out_shape=[
        jax.ShapeDtypeStruct((NP, D_HID // 2), jnp.float32),
        jax.ShapeDtypeStruct((NP, D_HID // 2), jnp.float32),
        jax.ShapeDtypeStruct((NP, 1), jnp.float32),
    ],
)


def _tcd_body(s0_ref, s1_ref, hsa_ref, hsb_ref, dis_ref, b1_ref, w2_ref,
              outa_ref, outb_ref):
    dis = dis_ref[...]
    s_full = jnp.concatenate([s0_ref[...], s1_ref[...]], axis=1)
    hs_full = jnp.concatenate([hsa_ref[...], hsb_ref[...]], axis=1)
    pre = (s_full + hs_full) * dis + b1_ref[...]
    h = jnp.maximum(pre, 0.0)
    h2 = jnp.dot(h, w2_ref[...], preferred_element_type=jnp.float32)
    row = (lax.broadcasted_iota(jnp.int32, (BLK, 1), 0)
           + pl.program_id(0) * BLK)
    hs2 = jnp.where(row < N, h2 * dis, 0.0)
    outa_ref[...] = hs2[:, :D_LAT // 2]
    outb_ref[...] = hs2[:, D_LAT // 2:]


_tcd = pl.pallas_call(
    _tcd_body,
    grid=(NP // BLK,),
    in_specs=[
        pl.BlockSpec((BLK, D_HID // 2), lambda i: (i, 0)),
        pl.BlockSpec((BLK, D_HID // 2), lambda i: (i, 0)),
        pl.BlockSpec((BLK, D_HID // 2), lambda i: (i, 0)),
        pl.BlockSpec((BLK, D_HID // 2), lambda i: (i, 0)),
        pl.BlockSpec((BLK, 1), lambda i: (i, 0)),
        pl.BlockSpec((1, D_HID), lambda i: (0, 0)),
        pl.BlockSpec((D_HID, D_LAT), lambda i: (0, 0)),
    ],
    out_specs=[
        pl.BlockSpec((BLK, D_LAT // 2), lambda i: (i, 0)),
        pl.BlockSpec((BLK, D_LAT // 2), lambda i: (i, 0)),
    ],
    out_shape=[
        jax.ShapeDtypeStruct((NP, D_LAT // 2), jnp.float32),
        jax.ShapeDtypeStruct((NP, D_LAT // 2), jnp.float32),
    ],
)


def _tcf_body(s0_ref, s1_ref, hsa_ref, hsb_ref, dis_ref, b2_ref, w1a_ref,
              w1b_ref, fb1_ref, a_ref, b_ref):
    s_full = jnp.concatenate([s0_ref[...], s1_ref[...]], axis=1)
    hs_full = jnp.concatenate([hsa_ref[...], hsb_ref[...]], axis=1)
    z = (s_full + hs_full) * dis_ref[...] + b2_ref[...]
    a_ref[...] = (jnp.dot(z, w1a_ref[...], preferred_element_type=jnp.float32)
                  + fb1_ref[...])
    b_ref[...] = jnp.dot(z, w1b_ref[...], preferred_element_type=jnp.float32)


_tcf = pl.pallas_call(
    _tcf_body,
    grid=(NP // BLK,),
    in_specs=[
        pl.BlockSpec((BLK, D_LAT // 2), lambda i: (i, 0)),
        pl.BlockSpec((BLK, D_LAT // 2), lambda i: (i, 0)),
        pl.BlockSpec((BLK, D_LAT // 2), lambda i: (i, 0)),
        pl.BlockSpec((BLK, D_LAT // 2), lambda i: (i, 0)),
        pl.BlockSpec((BLK, 1), lambda i: (i, 0)),
        pl.BlockSpec((1, D_LAT), lambda i: (0, 0)),
        pl.BlockSpec((D_LAT, D_LAT), lambda i: (0, 0)),
        pl.BlockSpec((D_LAT, D_LAT), lambda i: (0, 0)),
        pl.BlockSpec((1, D_LAT), lambda i: (0, 0)),
    ],
    out_specs=[
        pl.BlockSpec((BLK, D_LAT), lambda i: (i, 0)),
        pl.BlockSpec((BLK, D_LAT), lambda i: (i, 0)),
    ],
    out_shape=[
        jax.ShapeDtypeStruct((NP, D_LAT), jnp.float32),
        jax.ShapeDtypeStruct((NP, D_LAT), jnp.float32),
    ],
)


def _pad_edges(idx, nw):
    pad = N + jnp.arange(EP - E, dtype=jnp.int32) % (NP - N)
    return jnp.concatenate([idx.astype(jnp.int32), pad]).reshape(nw, -1, CH)


def kernel(x, edge_index, eval_edge_index, W1, b1, W2, b2, fcW1, fcb1,
           fcW2, fcb2):
    f32 = jnp.float32
    xp = jnp.pad(x, ((0, NP - N), (0, 0)))
    dst3 = _pad_edges(edge_index[0], NW)
    srcs = _pad_edges(edge_index[0], NS)
    dsts = _pad_edges(edge_index[1], NS)
    es3 = _pad_edges(eval_edge_index[0], NW)
    ed3 = _pad_edges(eval_edge_index[1], NW)

    deg2 = _deg_kernel(_pad_edges(edge_index[1], NW))
    hs1a, hs1b, dis = _tcb(xp, W1, deg2[0], deg2[1])
    s1 = _scatter128(hs1a, hs1b, srcs, dsts)
    hs2a, hs2b = _tcd(s1[0], s1[1], hs1a, hs1b, dis,
                      b1.reshape(1, D_HID), W2)
    s2 = _scatter64(hs2a, hs2b, srcs, dsts)
    A, B = _tcf(s2[0], s2[1], hs2a, hs2b, dis, b2.reshape(1, D_LAT),
                fcW1[:D_LAT], fcW1[D_LAT:], fcb1.reshape(1, D_LAT))
    wc = jnp.concatenate([fcW2.reshape(-1), fcb2.reshape(-1),
                          jnp.zeros((15,), f32)])
    scores = _scorer(A, B, es3, ed3, wc)
    return scores.reshape(-1)[:E]


# speedup vs baseline: 1.0060x; 1.0060x over previous
import functools

import jax
import jax.numpy as jnp
from jax import lax
from jax.experimental import pallas as pl
from jax.experimental.pallas import tpu as pltpu
from jax.experimental.pallas import tpu_sc as plsc

N = 10000
NP = 10240
E = 320000
EP = 327680
D_IN = 128
D_HID = 128
D_LAT = 64
NC = 2
NS = 16
NW = NC * NS
CH = 128
NCH = EP // (NW * CH)
RPT = NP // NS
NPA = NP
RPTA = NPA // NS
NCHS = EP // (NS * CH)
ZB = 64
DUMMY = N
BLK = 2048


def _sc_mesh():
    return plsc.VectorSubcoreMesh(core_axis_name="c", subcore_axis_name="s")


@functools.partial(
    pl.kernel,
    out_type=jax.ShapeDtypeStruct((NC, NP, 16), jnp.float32),
    mesh=_sc_mesh(),
    scratch_types=[
        pltpu.VMEM((NCH, CH), jnp.int32),
        pltpu.VMEM((CH, 16), jnp.float32),
        pltpu.VMEM((ZB, 16), jnp.float32),
        pltpu.VMEM_SHARED((NP, 16), jnp.float32),
    ],
    compiler_params=pltpu.CompilerParams(use_tc_tiling_on_sc=False),
)
def _deg_kernel(dst_hbm, out_hbm, dst_v, ones_v, zb_v, acc_sh):
    c = lax.axis_index("c")
    s = lax.axis_index("s")
    wid = s * NC + c
    pltpu.sync_copy(dst_hbm.at[wid], dst_v)
    one16 = jnp.ones((16,), jnp.float32)
    zero16 = jnp.zeros((16,), jnp.float32)

    def fill_ones(i, _):
        ones_v[i, :] = one16
        return 0

    lax.fori_loop(0, CH, fill_ones, 0)

    def fill_zero(i, _):
        zb_v[i, :] = zero16
        return 0

    lax.fori_loop(0, ZB, fill_zero, 0)

    base = s * RPT

    def zcopy(i, _):
        pltpu.sync_copy(zb_v, acc_sh.at[pl.ds(base + i * ZB, ZB)])
        return 0

    lax.fori_loop(0, RPT // ZB, zcopy, 0)
    plsc.subcore_barrier()

    def chunk(j, _):
        pltpu.sync_copy(ones_v, acc_sh.at[dst_v.at[j]], add=True)
        return 0

    lax.fori_loop(0, NCH, chunk, 0)
    plsc.subcore_barrier()
    pltpu.sync_copy(acc_sh.at[pl.ds(base, RPT)], out_hbm.at[c, pl.ds(base, RPT)])


def _make_scatter(DH):
    @functools.partial(
        pl.kernel,
        out_type=jax.ShapeDtypeStruct((NC, NP, DH), jnp.float32),
        mesh=_sc_mesh(),
        scratch_types=[
            pltpu.VMEM((NCHS, CH), jnp.int32),
            pltpu.VMEM((NCHS, CH), jnp.int32),
            [pltpu.VMEM((CH, DH), jnp.float32)] * 4,
            [pltpu.VMEM((CH,), jnp.int32)] * 4,
            pltpu.VMEM((CH,), jnp.int32),
            pltpu.VMEM((ZB, DH), jnp.float32),
            pltpu.VMEM_SHARED((NPA, DH), jnp.float32),
            [pltpu.SemaphoreType.DMA] * 4,
        ],
        compiler_params=pltpu.CompilerParams(use_tc_tiling_on_sc=False),
    )
    def _scatter(hsa_hbm, hsb_hbm, src_hbm, dst_hbm, out_hbm, src_v, dst_v,
                 rows_v, gidx, sidx, zb_v, acc_sh, gsem):
        c = lax.axis_index("c")
        s = lax.axis_index("s")
        pltpu.sync_copy(src_hbm.at[s], src_v)
        pltpu.sync_copy(dst_hbm.at[s], dst_v)
        zero16 = jnp.zeros((16,), jnp.float32)

        def fz(i, _):
            for l in range(DH // 16):
                zb_v[i, pl.ds(l * 16, 16)] = zero16
            return 0

        lax.fori_loop(0, ZB, fz, 0)
        base = s * RPTA

        def zc(i, _):
            pltpu.sync_copy(zb_v, acc_sh.at[pl.ds(base + i * ZB, ZB)])
            return 0

        lax.fori_loop(0, RPTA // ZB, zc, 0)
        if RPTA % ZB:
            pltpu.sync_copy(zb_v.at[pl.ds(0, RPTA % ZB)],
                            acc_sh.at[pl.ds(base + (RPTA // ZB) * ZB,
                                            RPTA % ZB)])
        plsc.subcore_barrier()

        def _cpidx(dst_ref, src_row):
            for k in range(CH // 16):
                dst_ref[pl.ds(k * 16, 16)] = src_row[pl.ds(k * 16, 16)]

        def _gather(b):
            @pl.when(c == 0)
            def _():
                pltpu.async_copy(hsa_hbm.at[gidx[b]], rows_v[b], gsem[b])

            @pl.when(c == 1)
            def _():
                pltpu.async_copy(hsb_hbm.at[gidx[b]], rows_v[b], gsem[b])

        for b in range(3):
            _cpidx(gidx[b], src_v.at[b])
            _gather(b)

        def rnd(jo, _):
            for b in range(4):
                j = jo * 4 + b
                pltpu.make_async_copy(hsa_hbm.at[gidx[b]], rows_v[b],
                                      gsem[b]).wait()

                @pl.when(j + 3 < NCHS)
                def _(b=b, j=j):
                    nb = (b + 3) % 4
                    _cpidx(gidx[nb], src_v.at[j + 3])
                    _gather(nb)

                _cpidx(sidx, dst_v.at[j])
                pltpu.sync_copy(rows_v[b], acc_sh.at[sidx], add=True)
            return 0

        lax.fori_loop(0, NCHS // 4, rnd, 0)
        plsc.subcore_barrier()
        pltpu.sync_copy(acc_sh.at[pl.ds(base, RPTA)],
                        out_hbm.at[c, pl.ds(base, RPTA)])

    return _scatter


_scatter128 = _make_scatter(D_HID // 2)
_scatter64 = _make_scatter(D_LAT // 2)


@functools.partial(
    pl.kernel,
    out_type=jax.ShapeDtypeStruct((NW, NCH, CH), jnp.float32),
    mesh=_sc_mesh(),
    scratch_types=[
        pltpu.VMEM((NCH, CH), jnp.int32),
        pltpu.VMEM((NCH, CH), jnp.int32),
        [pltpu.VMEM((CH, D_LAT), jnp.float32)] * 2,
        [pltpu.VMEM((CH, D_LAT), jnp.float32)] * 2,
        [pltpu.VMEM((CH,), jnp.int32)] * 2,
        [pltpu.VMEM((CH,), jnp.int32)] * 2,
        [pltpu.VMEM((CH,), jnp.float32)] * 2,
        pltpu.VMEM((80,), jnp.float32),
        [pltpu.SemaphoreType.DMA] * 2,
        [pltpu.SemaphoreType.DMA] * 2,
        [pltpu.SemaphoreType.DMA] * 2,
    ],
    compiler_params=pltpu.CompilerParams(use_tc_tiling_on_sc=False,
                                         needs_layout_passes=False),
)
def _scorer(a_hbm, b_hbm, esrc_hbm, edst_hbm, wc_hbm, out_hbm,
            src_v, dst_v, ar_v, br_v, si_v, di_v, res_v, wc_v,
            sema, semb, semo):
    c = lax.axis_index("c")
    s = lax.axis_index("s")
    wid = s * NC + c
    pltpu.sync_copy(esrc_hbm.at[wid], src_v)
    pltpu.sync_copy(edst_hbm.at[wid], dst_v)
    pltpu.sync_copy(wc_hbm, wc_v)
    lane = lax.broadcasted_iota(jnp.int32, (16,), 0)
    b2s = plsc.load_gather(wc_v, [jnp.full((16,), 64, jnp.int32)])
    wregs = [wc_v[pl.ds(k * 16, 16)] for k in range(4)]
    eqm = [lane == u for u in range(16)]

    def _cpidx(dst_ref, src_row):
        for k in range(CH // 16):
            dst_ref[pl.ds(k * 16, 16)] = src_row[pl.ds(k * 16, 16)]

    NB = 2
    for b in range(NB):
        _cpidx(si_v[b], src_v.at[b])
        _cpidx(di_v[b], dst_v.at[b])
        pltpu.async_copy(a_hbm.at[si_v[b]], ar_v[b], sema[b])
        pltpu.async_copy(b_hbm.at[di_v[b]], br_v[b], semb[b])

    def rnd(jo, _):
        for b in range(NB):
            j = jo * NB + b
            pltpu.make_async_copy(a_hbm.at[si_v[b]], ar_v[b],
                                  sema[b]).wait()
            pltpu.make_async_copy(b_hbm.at[di_v[b]], br_v[b],
                                  semb[b]).wait()
            arb = ar_v[b]
            brb = br_v[b]

            @pl.when(jo > 0)
            def _(b=b):
                pltpu.make_async_copy(res_v[b], out_hbm.at[wid, 0],
                                      semo[b]).wait()

            def group(g, _, arb=arb, brb=brb, b=b):
                vres = b2s
                for u in range(16):
                    e = g * 16 + u
                    t = None
                    for k in range(4):
                        av = arb[e, pl.ds(k * 16, 16)]
                        bv = brb[e, pl.ds(k * 16, 16)]
                        tk = jnp.maximum(av + bv, 0.0) * wregs[k]
                        t = tk if t is None else t + tk
                    s = jnp.sum(t)
                    vres = jnp.where(eqm[u], jnp.full((16,), s, jnp.float32),
                                     vres)
                res_v[b][pl.ds(g * 16, 16)] = 1.0 / (1.0 + jnp.exp(-vres))
                return 0

            lax.fori_loop(0, CH // 16, group, 0)
            pltpu.async_copy(res_v[b], out_hbm.at[wid, j], semo[b])

            @pl.when(jo + 1 < NCH // NB)
            def _(b=b, jo=jo):
                j2 = (jo + 1) * NB + b
                _cpidx(si_v[b], src_v.at[j2])
                _cpidx(di_v[b], dst_v.at[j2])
                pltpu.async_copy(a_hbm.at[si_v[b]], ar_v[b], sema[b])
                pltpu.async_copy(b_hbm.at[di_v[b]], br_v[b], semb[b])

        return 0

    lax.fori_loop(0, NCH // NB, rnd, 0)
    for b in range(NB):
        pltpu.make_async_copy(res_v[b], out_hbm.at[wid, 0], semo[b]).wait()


def _tcb_body(x_ref, w1_ref, d0_ref, d1_ref, hsa_ref, hsb_ref, dis_ref):
    deg = d0_ref[:, 0:1] + d1_ref[:, 0:1] + 1.0
    dis = lax.rsqrt(deg)
    h1 = jnp.dot(x_ref[...], w1_ref[...], preferred_element_type=jnp.float32)
    hs = h1 * dis
    hsa_ref[...] = hs[:, :D_HID // 2]
    hsb_ref[...] = hs[:, D_HID // 2:]
    dis_ref[...] = dis


_tcb = pl.pallas_call(
    _tcb_body,
    grid=(NP // BLK,),
    in_specs=[
        pl.BlockSpec((BLK, D_IN), lambda i: (i, 0)),
        pl.BlockSpec((D_IN, D_HID), lambda i: (0, 0)),
        pl.BlockSpec((BLK, 16), lambda i: (i, 0)),
        pl.BlockSpec((BLK, 16), lambda i: (i, 0)),
    ],
    out_specs=[
        pl.BlockSpec((BLK, D_HID // 2), lambda i: (i, 0)),
        pl.BlockSpec((BLK, D_HID // 2), lambda i: (i, 0)),
        pl.BlockSpec((BLK, 1), lambda i: (i, 0)),
    ],
    out_shape=[
        jax.ShapeDtypeStruct((NP, D_HID // 2), jnp.float32),
        jax.ShapeDtypeStruct((NP, D_HID // 2), jnp.float32),
        jax.ShapeDtypeStruct((NP, 1), jnp.float32),
    ],
)


def _tcd_body(s0_ref, s1_ref, hsa_ref, hsb_ref, dis_ref, b1_ref, w2_ref,
              outa_ref, outb_ref):
    dis = dis_ref[...]
    s_full = jnp.concatenate([s0_ref[...], s1_ref[...]], axis=1)
    hs_full = jnp.concatenate([hsa_ref[...], hsb_ref[...]], axis=1)
    pre = (s_full + hs_full) * dis + b1_ref[...]
    h = jnp.maximum(pre, 0.0)
    h2 = jnp.dot(h, w2_ref[...], preferred_element_type=jnp.float32)
    row = (lax.broadcasted_iota(jnp.int32, (BLK, 1), 0)
           + pl.program_id(0) * BLK)
    hs2 = jnp.where(row < N, h2 * dis, 0.0)
    outa_ref[...] = hs2[:, :D_LAT // 2]
    outb_ref[...] = hs2[:, D_LAT // 2:]


_tcd = pl.pallas_call(
    _tcd_body,
    grid=(NP // BLK,),
    in_specs=[
        pl.BlockSpec((BLK, D_HID // 2), lambda i: (i, 0)),
        pl.BlockSpec((BLK, D_HID // 2), lambda i: (i, 0)),
        pl.BlockSpec((BLK, D_HID // 2), lambda i: (i, 0)),
        pl.BlockSpec((BLK, D_HID // 2), lambda i: (i, 0)),
        pl.BlockSpec((BLK, 1), lambda i: (i, 0)),
        pl.BlockSpec((1, D_HID), lambda i: (0, 0)),
        pl.BlockSpec((D_HID, D_LAT), lambda i: (0, 0)),
    ],
    out_specs=[
        pl.BlockSpec((BLK, D_LAT // 2), lambda i: (i, 0)),
        pl.BlockSpec((BLK, D_LAT // 2), lambda i: (i, 0)),
    ],
    out_shape=[
        jax.ShapeDtypeStruct((NP, D_LAT // 2), jnp.float32),
        jax.ShapeDtypeStruct((NP, D_LAT // 2), jnp.float32),
    ],
)


def _tcf_body(s0_ref, s1_ref, hsa_ref, hsb_ref, dis_ref, b2_ref, w1a_ref,
              w1b_ref, fb1_ref, a_ref, b_ref):
    s_full = jnp.concatenate([s0_ref[...], s1_ref[...]], axis=1)
    hs_full = jnp.concatenate([hsa_ref[...], hsb_ref[...]], axis=1)
    z = (s_full + hs_full) * dis_ref[...] + b2_ref[...]
    a_ref[...] = (jnp.dot(z, w1a_ref[...], preferred_element_type=jnp.float32)
                  + fb1_ref[...])
    b_ref[...] = jnp.dot(z, w1b_ref[...], preferred_element_type=jnp.float32)


_tcf = pl.pallas_call(
    _tcf_body,
    grid=(NP // BLK,),
    in_specs=[
        pl.BlockSpec((BLK, D_LAT // 2), lambda i: (i, 0)),
        pl.BlockSpec((BLK, D_LAT // 2), lambda i: (i, 0)),
        pl.BlockSpec((BLK, D_LAT // 2), lambda i: (i, 0)),
        pl.BlockSpec((BLK, D_LAT // 2), lambda i: (i, 0)),
        pl.BlockSpec((BLK, 1), lambda i: (i, 0)),
        pl.BlockSpec((1, D_LAT), lambda i: (0, 0)),
        pl.BlockSpec((D_LAT, D_LAT), lambda i: (0, 0)),
        pl.BlockSpec((D_LAT, D_LAT), lambda i: (0, 0)),
        pl.BlockSpec((1, D_LAT), lambda i: (0, 0)),
    ],
    out_specs=[
        pl.BlockSpec((BLK, D_LAT), lambda i: (i, 0)),
        pl.BlockSpec((BLK, D_LAT), lambda i: (i, 0)),
    ],
    out_shape=[
        jax.ShapeDtypeStruct((NP, D_LAT), jnp.float32),
        jax.ShapeDtypeStruct((NP, D_LAT), jnp.float32),
    ],
)


def _pad_edges(idx, nw):
    pad = N + jnp.arange(EP - E, dtype=jnp.int32) % (NP - N)
    return jnp.concatenate([idx.astype(jnp.int32), pad]).reshape(nw, -1, CH)


def kernel(x, edge_index, eval_edge_index, W1, b1, W2, b2, fcW1, fcb1,
           fcW2, fcb2):
    f32 = jnp.float32
    xp = jnp.pad(x, ((0, NP - N), (0, 0)))
    srcs = _pad_edges(edge_index[0], NS)
    dsts = _pad_edges(edge_index[1], NS)
    es3 = _pad_edges(eval_edge_index[0], NW)
    ed3 = _pad_edges(eval_edge_index[1], NW)

    deg2 = _deg_kernel(_pad_edges(edge_index[1], NW))
    hs1a, hs1b, dis = _tcb(xp, W1, deg2[0], deg2[1])
    s1 = _scatter128(hs1a, hs1b, srcs, dsts)
    hs2a, hs2b = _tcd(s1[0], s1[1], hs1a, hs1b, dis,
                      b1.reshape(1, D_HID), W2)
    s2 = _scatter64(hs2a, hs2b, srcs, dsts)
    A, B = _tcf(s2[0], s2[1], hs2a, hs2b, dis, b2.reshape(1, D_LAT),
                fcW1[:D_LAT], fcW1[D_LAT:], fcb1.reshape(1, D_LAT))
    wc = jnp.concatenate([fcW2.reshape(-1), fcb2.reshape(-1),
                          jnp.zeros((15,), f32)])
    scores = _scorer(A, B, es3, ed3, wc)
    return scores.reshape(-1)[:E]
